# baseline (speedup 1.0000x reference)
import jax
import jax.numpy as jnp
from jax.experimental import pallas as pl
from jax.experimental.pallas import tpu as pltpu

VOCAB = 32768
MASK_ID = 32767
ROWS_PER_BLOCK = 128
CHUNK = 4096


def _blend_kernel(z_ref, logits_hbm, out_ref, scratch, sem):
    i = pl.program_id(0)
    z = z_ref[:, :]
    r, c = out_ref.shape
    any_masked = jnp.any(z == MASK_ID)

    @pl.when(any_masked)
    def _():
        nchunks = c // CHUNK

        def chunk_body(j, carry):
            cp = pltpu.make_async_copy(
                logits_hbm.at[pl.ds(i * r, r), pl.ds(j * CHUNK, CHUNK)],
                scratch,
                sem,
            )
            cp.start()
            cp.wait()
            col = j * CHUNK + jax.lax.broadcasted_iota(
                jnp.int32, (r, CHUNK), 1
            )
            onehot = jnp.where(col == z, jnp.float32(1e9), jnp.float32(0.0))
            lg = jnp.where(
                col == MASK_ID, jnp.float32(-jnp.inf), scratch[:, :]
            )
            out_ref[:, pl.ds(j * CHUNK, CHUNK)] = jnp.where(
                z == MASK_ID, lg, onehot
            )
            return carry

        jax.lax.fori_loop(0, nchunks, chunk_body, 0)

    @pl.when(jnp.logical_not(any_masked))
    def _():
        out_ref[:, :] = jnp.zeros((r, c), jnp.float32)

        pass


def kernel(logits, z_t):
    b, t, v = logits.shape
    n = b * t
    lf = logits.reshape(n, v)
    zf = z_t.reshape(n, 1)
    r = ROWS_PER_BLOCK
    out = pl.pallas_call(
        _blend_kernel,
        grid=(n // r,),
        in_specs=[
            pl.BlockSpec((r, 1), lambda i: (i, 0)),
            pl.BlockSpec(memory_space=pl.ANY),
        ],
        out_specs=pl.BlockSpec((r, v), lambda i: (i, 0)),
        out_shape=jax.ShapeDtypeStruct((n, v), jnp.float32),
        scratch_shapes=[
            pltpu.VMEM((r, CHUNK), jnp.float32),
            pltpu.SemaphoreType.DMA,
        ],
    )(zf, lf)
    return out.reshape(b, t, v)


# speedup vs baseline: 1.0036x; 1.0036x over previous
import jax
import jax.numpy as jnp
from jax.experimental import pallas as pl
from jax.experimental.pallas import tpu as pltpu

VOCAB = 32768
MASK_ID = 32767
ROWS_PER_BLOCK = 64
CHUNK = 4096


def _blend_kernel(z_ref, logits_hbm, out_ref, scratch, sem):
    i = pl.program_id(0)
    z = z_ref[:, :]
    r, c = out_ref.shape
    any_masked = jnp.any(z == MASK_ID)

    @pl.when(any_masked)
    def _():
        nchunks = c // CHUNK

        def chunk_body(j, carry):
            cp = pltpu.make_async_copy(
                logits_hbm.at[pl.ds(i * r, r), pl.ds(j * CHUNK, CHUNK)],
                scratch,
                sem,
            )
            cp.start()
            cp.wait()
            col = j * CHUNK + jax.lax.broadcasted_iota(
                jnp.int32, (r, CHUNK), 1
            )
            onehot = jnp.where(col == z, jnp.float32(1e9), jnp.float32(0.0))
            lg = jnp.where(
                col == MASK_ID, jnp.float32(-jnp.inf), scratch[:, :]
            )
            out_ref[:, pl.ds(j * CHUNK, CHUNK)] = jnp.where(
                z == MASK_ID, lg, onehot
            )
            return carry

        jax.lax.fori_loop(0, nchunks, chunk_body, 0)

    @pl.when(jnp.logical_not(any_masked))
    def _():
        out_ref[:, :] = jnp.zeros((r, c), jnp.float32)

        col = jax.lax.broadcasted_iota(jnp.int32, (r, c), 1)
        out_ref[:, :] = jnp.where(col == z, jnp.float32(1e9), jnp.float32(0.0))


def kernel(logits, z_t):
    b, t, v = logits.shape
    n = b * t
    lf = logits.reshape(n, v)
    zf = z_t.reshape(n, 1)
    r = ROWS_PER_BLOCK
    out = pl.pallas_call(
        _blend_kernel,
        grid=(n // r,),
        in_specs=[
            pl.BlockSpec((r, 1), lambda i: (i, 0)),
            pl.BlockSpec(memory_space=pl.ANY),
        ],
        out_specs=pl.BlockSpec((r, v), lambda i: (i, 0)),
        out_shape=jax.ShapeDtypeStruct((n, v), jnp.float32),
        scratch_shapes=[
            pltpu.VMEM((r, CHUNK), jnp.float32),
            pltpu.SemaphoreType.DMA,
        ],
    )(zf, lf)
    return out.reshape(b, t, v)
